# Initial kernel scaffold; baseline (speedup 1.0000x reference)
import jax
import jax.numpy as jnp
from jax.experimental import pallas as pl


def kernel(hidden_states, W_mask, b_mask, W_delta, b_delta, num_selected):
    raise NotImplementedError("write your pallas kernel here")


# speedup vs baseline: 1.5477x; 1.5477x over previous
import functools

import jax
import jax.numpy as jnp
from jax.experimental import pallas as pl
from jax.experimental.pallas import tpu as pltpu


def _edit_head_kernel(h_ref, wm_ref, bm_ref, wd_ref, bd_ref,
                      mask_out_ref, delta_out_ref, acc_ref,
                      *, n_blocks, seq_len, num_selected_static):
    i = pl.program_id(0)
    h = h_ref[...]

    part = jnp.sum(h, axis=1)

    @pl.when(i == 0)
    def _init():
        acc_ref[...] = part

    @pl.when(i > 0)
    def _accum():
        acc_ref[...] += part

    @pl.when(i == n_blocks - 1)
    def _finish():
        last_hidden = h[:, -1, :]
        mask_out_ref[...] = (
            jnp.dot(last_hidden, wm_ref[...],
                    preferred_element_type=jnp.float32) + bm_ref[...]
        )
        mean_h = acc_ref[...] * (1.0 / seq_len)
        delta_row = (
            jnp.dot(mean_h, wd_ref[...],
                    preferred_element_type=jnp.float32) + bd_ref[...]
        )
        delta_out_ref[...] = jnp.broadcast_to(
            delta_row[:, None, :],
            (delta_row.shape[0], num_selected_static, delta_row.shape[1]),
        )


_NUM_SELECTED_STATIC = 256


@jax.jit
def _edit_head(hidden_states, W_mask, b_mask, W_delta, b_delta):
    B, S, H = hidden_states.shape
    M = W_mask.shape[1]
    D = W_delta.shape[1]
    num_selected = _NUM_SELECTED_STATIC
    S_BLK = 256
    n_blocks = S // S_BLK

    mask_flat, edit_delta = pl.pallas_call(
        functools.partial(
            _edit_head_kernel,
            n_blocks=n_blocks,
            seq_len=S,
            num_selected_static=num_selected,
        ),
        grid=(n_blocks,),
        in_specs=[
            pl.BlockSpec((B, S_BLK, H), lambda i: (0, i, 0)),
            pl.BlockSpec((H, M), lambda i: (0, 0)),
            pl.BlockSpec((M,), lambda i: (0,)),
            pl.BlockSpec((H, D), lambda i: (0, 0)),
            pl.BlockSpec((D,), lambda i: (0,)),
        ],
        out_specs=[
            pl.BlockSpec((B, M), lambda i: (0, 0)),
            pl.BlockSpec((B, num_selected, D), lambda i: (0, 0, 0)),
        ],
        out_shape=[
            jax.ShapeDtypeStruct((B, M), jnp.float32),
            jax.ShapeDtypeStruct((B, num_selected, D), jnp.float32),
        ],
        scratch_shapes=[pltpu.VMEM((B, H), jnp.float32)],
    )(hidden_states, W_mask, b_mask, W_delta, b_delta)

    mask_size = int(round(M ** 0.5))
    sparse_mask = mask_flat.reshape(B, mask_size, mask_size)
    return sparse_mask, edit_delta


def kernel(hidden_states, W_mask, b_mask, W_delta, b_delta, num_selected):
    del num_selected
    return _edit_head(hidden_states, W_mask, b_mask, W_delta, b_delta)


# speedup vs baseline: 1.6019x; 1.0350x over previous
import functools

import jax
import jax.numpy as jnp
from jax.experimental import pallas as pl
from jax.experimental.pallas import tpu as pltpu


def _edit_head_kernel(h_ref, wm_ref, bm_ref, wd_ref, bd_ref,
                      mask_out_ref, delta_out_ref, acc_ref,
                      *, n_blocks, seq_len, num_selected_static):
    i = pl.program_id(0)
    h = h_ref[...]

    part = jnp.sum(h, axis=1)

    @pl.when(i == 0)
    def _init():
        acc_ref[...] = part

    @pl.when(i > 0)
    def _accum():
        acc_ref[...] += part

    @pl.when(i == n_blocks - 1)
    def _finish():
        last_hidden = h[:, -1, :]
        mask_out_ref[...] = (
            jnp.dot(last_hidden, wm_ref[...],
                    preferred_element_type=jnp.float32) + bm_ref[...]
        )
        mean_h = acc_ref[...] * (1.0 / seq_len)
        delta_row = (
            jnp.dot(mean_h, wd_ref[...],
                    preferred_element_type=jnp.float32) + bd_ref[...]
        )
        delta_out_ref[...] = jnp.broadcast_to(
            delta_row[:, None, :],
            (delta_row.shape[0], num_selected_static, delta_row.shape[1]),
        )


_NUM_SELECTED_STATIC = 256


@jax.jit
def _edit_head(hidden_states, W_mask, b_mask, W_delta, b_delta):
    B, S, H = hidden_states.shape
    M = W_mask.shape[1]
    D = W_delta.shape[1]
    num_selected = _NUM_SELECTED_STATIC
    S_BLK = 512
    n_blocks = S // S_BLK

    mask_flat, edit_delta = pl.pallas_call(
        functools.partial(
            _edit_head_kernel,
            n_blocks=n_blocks,
            seq_len=S,
            num_selected_static=num_selected,
        ),
        grid=(n_blocks,),
        in_specs=[
            pl.BlockSpec((B, S_BLK, H), lambda i: (0, i, 0)),
            pl.BlockSpec((H, M), lambda i: (0, 0)),
            pl.BlockSpec((M,), lambda i: (0,)),
            pl.BlockSpec((H, D), lambda i: (0, 0)),
            pl.BlockSpec((D,), lambda i: (0,)),
        ],
        out_specs=[
            pl.BlockSpec((B, M), lambda i: (0, 0)),
            pl.BlockSpec((B, num_selected, D), lambda i: (0, 0, 0)),
        ],
        out_shape=[
            jax.ShapeDtypeStruct((B, M), jnp.float32),
            jax.ShapeDtypeStruct((B, num_selected, D), jnp.float32),
        ],
        scratch_shapes=[pltpu.VMEM((B, H), jnp.float32)],
    )(hidden_states, W_mask, b_mask, W_delta, b_delta)

    mask_size = int(round(M ** 0.5))
    sparse_mask = mask_flat.reshape(B, mask_size, mask_size)
    return sparse_mask, edit_delta


def kernel(hidden_states, W_mask, b_mask, W_delta, b_delta, num_selected):
    del num_selected
    return _edit_head(hidden_states, W_mask, b_mask, W_delta, b_delta)


# speedup vs baseline: 1.6132x; 1.0070x over previous
import functools

import jax
import jax.numpy as jnp
from jax.experimental import pallas as pl
from jax.experimental.pallas import tpu as pltpu


def _edit_head_kernel(h_ref, wm_ref, bm_ref, wd_ref, bd_ref,
                      mask_out_ref, delta_out_ref,
                      *, seq_len, num_selected_static):
    h = h_ref[0]

    last_hidden = h[-1:, :]
    mask_out_ref[0] = (
        jnp.dot(last_hidden, wm_ref[...],
                preferred_element_type=jnp.float32) + bm_ref[...]
    )

    mean_h = (jnp.sum(h, axis=0, keepdims=True) * (1.0 / seq_len))
    delta_row = (
        jnp.dot(mean_h, wd_ref[...],
                preferred_element_type=jnp.float32) + bd_ref[...]
    )
    delta_out_ref[...] = jnp.broadcast_to(
        delta_row[:, None, :], (1, num_selected_static, delta_row.shape[1])
    )


_NUM_SELECTED_STATIC = 256


@jax.jit
def _edit_head(hidden_states, W_mask, b_mask, W_delta, b_delta):
    B, S, H = hidden_states.shape
    M = W_mask.shape[1]
    D = W_delta.shape[1]
    num_selected = _NUM_SELECTED_STATIC

    mask_flat, edit_delta = pl.pallas_call(
        functools.partial(
            _edit_head_kernel,
            seq_len=S,
            num_selected_static=num_selected,
        ),
        grid=(B,),
        in_specs=[
            pl.BlockSpec((1, S, H), lambda i: (i, 0, 0)),
            pl.BlockSpec((H, M), lambda i: (0, 0)),
            pl.BlockSpec((M,), lambda i: (0,)),
            pl.BlockSpec((H, D), lambda i: (0, 0)),
            pl.BlockSpec((D,), lambda i: (0,)),
        ],
        out_specs=[
            pl.BlockSpec((1, 1, M), lambda i: (i, 0, 0)),
            pl.BlockSpec((1, num_selected, D), lambda i: (i, 0, 0)),
        ],
        out_shape=[
            jax.ShapeDtypeStruct((B, 1, M), jnp.float32),
            jax.ShapeDtypeStruct((B, num_selected, D), jnp.float32),
        ],
        compiler_params=pltpu.CompilerParams(
            dimension_semantics=("arbitrary",),
        ),
    )(hidden_states, W_mask, b_mask, W_delta, b_delta)

    mask_size = int(round(M ** 0.5))
    sparse_mask = mask_flat.reshape(B, mask_size, mask_size)
    return sparse_mask, edit_delta


def kernel(hidden_states, W_mask, b_mask, W_delta, b_delta, num_selected):
    del num_selected
    return _edit_head(hidden_states, W_mask, b_mask, W_delta, b_delta)
